# Initial kernel scaffold; baseline (speedup 1.0000x reference)
import jax
import jax.numpy as jnp
from jax.experimental import pallas as pl


def kernel(x, neighbor_idx, down_w, down_b, up_w, up_b, gate_w, gate_b):
    raise NotImplementedError("write your pallas kernel here")


# speedup vs baseline: 2.1471x; 2.1471x over previous
import functools

import jax
import jax.numpy as jnp
from jax import lax
from jax.experimental import pallas as pl
from jax.experimental.pallas import tpu as pltpu
from jax.experimental.pallas import tpu_sc as plsc

B, T, D = 4, 8192, 512
BOT = 64
N = B * T

_CH = 128


@functools.cache
def _make_sc_gather():
    info = plsc.get_sparse_core_info()
    nc, ns = info.num_cores, info.num_subcores
    nw = nc * ns
    rpw = N // nw
    chunks = rpw // _CH
    mesh = plsc.VectorSubcoreMesh(core_axis_name="c", subcore_axis_name="s")

    @functools.partial(
        pl.kernel,
        mesh=mesh,
        out_type=jax.ShapeDtypeStruct((N, D), jnp.float32),
        scratch_types=[
            pltpu.VMEM((_CH,), jnp.int32),
            pltpu.VMEM((_CH, D), jnp.float32),
            pltpu.SemaphoreType.DMA,
        ],
    )
    def sc_gather(x_hbm, idx_hbm, out_hbm, idx_v, rows_v, sem):
        wid = lax.axis_index("s") * nc + lax.axis_index("c")
        base = wid * rpw
        for j in range(chunks):
            off = base + j * _CH
            pltpu.sync_copy(idx_hbm.at[pl.ds(off, _CH)], idx_v)
            pltpu.async_copy(x_hbm.at[idx_v], rows_v, sem).wait()
            pltpu.sync_copy(rows_v, out_hbm.at[pl.ds(off, _CH)])

    return sc_gather


_BT = 1024
_GRID = N // _BT


def _tc_body(x_ref, agg_ref, idxf_ref, dw_ref, db_ref, uw_ref, ub_ref,
             gw_ref, gb_ref, out_ref):
    xb = x_ref[...]
    ab = agg_ref[...]
    h = jnp.maximum(
        jnp.dot(ab, dw_ref[...], preferred_element_type=jnp.float32)
        + db_ref[...], 0.0)
    z = jnp.dot(h, uw_ref[...], preferred_element_type=jnp.float32) + ub_ref[...]
    garg = (jnp.dot(xb, gw_ref[:D, :], preferred_element_type=jnp.float32)
            + jnp.dot(ab, gw_ref[D:, :], preferred_element_type=jnp.float32)
            + gb_ref[...])
    g = jax.nn.sigmoid(garg)
    r0 = pl.program_id(0) * _BT
    rows = r0 + lax.broadcasted_iota(jnp.int32, (_BT, 1), 0)
    t = rows % T
    nbr = idxf_ref[...] - (rows // T) * T
    mask = ((t >= 1) & (nbr != 0)).astype(jnp.float32)
    out_ref[...] = xb + (g * z) * mask


def _tc_dense():
    return pl.pallas_call(
        _tc_body,
        grid=(_GRID,),
        in_specs=[
            pl.BlockSpec((_BT, D), lambda i: (i, 0)),
            pl.BlockSpec((_BT, D), lambda i: (i, 0)),
            pl.BlockSpec((_BT, 1), lambda i: (i, 0)),
            pl.BlockSpec((D, BOT), lambda i: (0, 0)),
            pl.BlockSpec((1, BOT), lambda i: (0, 0)),
            pl.BlockSpec((BOT, D), lambda i: (0, 0)),
            pl.BlockSpec((1, D), lambda i: (0, 0)),
            pl.BlockSpec((2 * D, D), lambda i: (0, 0)),
            pl.BlockSpec((1, D), lambda i: (0, 0)),
        ],
        out_specs=pl.BlockSpec((_BT, D), lambda i: (i, 0)),
        out_shape=jax.ShapeDtypeStruct((N, D), jnp.float32),
    )


def kernel(x, neighbor_idx, down_w, down_b, up_w, up_b, gate_w, gate_b):
    idx = neighbor_idx[:, 0]
    idxf = ((jnp.arange(B, dtype=jnp.int32) * T)[:, None]
            + idx[None, :]).reshape(N)
    x2 = x.reshape(N, D)
    agg = _make_sc_gather()(x2, idxf)
    out = _tc_dense()(
        x2, agg, idxf.reshape(N, 1),
        down_w, down_b.reshape(1, BOT), up_w, up_b.reshape(1, D),
        gate_w, gate_b.reshape(1, D))
    return out.reshape(B, T, D)
